# baseline (speedup 1.0000x reference)
import functools

import jax
import jax.numpy as jnp
from jax import lax
from jax.experimental import pallas as pl
from jax.experimental.pallas import tpu as pltpu
from jax.experimental.pallas import tpu_sc as plsc

NCENTROIDS = 1024
NCODEBOOKS = 16
SUBVECT = 256
BATCH = 4096

TB = 256
NB = BATCH // TB


def _mm_argmax_kernel(x_ref, cb_ref, out_ref):
    cols = []
    for c in range(NCODEBOOKS):
        resp = lax.dot_general(
            x_ref[:, c, :], cb_ref[c], (((1,), (1,)), ((), ())),
            preferred_element_type=jnp.float32,
        )
        code = jnp.argmax(resp, axis=1).astype(jnp.int32)
        cols.append((code + c * NCENTROIDS)[:, None])
    out_ref[...] = jnp.concatenate(cols, axis=1)


def _compute_codes_part(inputs, centroids, tile0, ntiles):
    return pl.pallas_call(
        _mm_argmax_kernel,
        grid=(ntiles,),
        in_specs=[
            pl.BlockSpec(
                (TB, NCODEBOOKS, SUBVECT), lambda b: (b + tile0, 0, 0)
            ),
            pl.BlockSpec(
                (NCODEBOOKS, NCENTROIDS, SUBVECT), lambda b: (0, 0, 0)
            ),
        ],
        out_specs=pl.BlockSpec((TB, NCODEBOOKS), lambda b: (b, 0)),
        out_shape=jax.ShapeDtypeStruct((ntiles * TB, NCODEBOOKS), jnp.int32),
    )(inputs, centroids)


def _make_gather(row0, nrows):
    info = plsc.get_sparse_core_info()
    nc, ns = info.num_cores, info.num_subcores
    nw = nc * ns
    rows_per_w = nrows // nw
    b_per_w = rows_per_w * NCODEBOOKS
    n_ch = 16
    ch = b_per_w // n_ch
    assert b_per_w % n_ch == 0 and ch % 8 == 0
    nbuf = 4
    mesh = plsc.VectorSubcoreMesh(core_axis_name="c", subcore_axis_name="s")

    @functools.partial(
        pl.kernel,
        mesh=mesh,
        out_type=(),
        scratch_types=[
            pltpu.VMEM((rows_per_w, NCODEBOOKS), jnp.int32),
            pltpu.VMEM((b_per_w,), jnp.int32),
            [pltpu.VMEM((ch, SUBVECT), jnp.float32) for _ in range(nbuf)],
            [pltpu.SemaphoreType.DMA for _ in range(nbuf)],
            [pltpu.SemaphoreType.DMA for _ in range(nbuf)],
        ],
    )
    def gather(table_hbm, idx_hbm, out_hbm, idx2d, idx_v, rows, gsem, wsem):
        wid = lax.axis_index("s") * nc + lax.axis_index("c")
        half_base = row0 * NCODEBOOKS
        pltpu.sync_copy(idx_hbm.at[pl.ds(wid * rows_per_w, rows_per_w), :], idx2d)

        def fl(r, _):
            idx_v[pl.ds(r * NCODEBOOKS, NCODEBOOKS)] = idx2d[r]
            return 0

        lax.fori_loop(0, rows_per_w, fl, 0)

        base = half_base + wid * b_per_w

        def start_g(j, t):
            pltpu.async_copy(
                table_hbm.at[idx_v.at[pl.ds(j * ch, ch)]], rows[t], gsem[t]
            )

        def wait_g(j, t):
            pltpu.make_async_copy(
                table_hbm.at[idx_v.at[pl.ds(j * ch, ch)]], rows[t], gsem[t]
            ).wait()

        def start_w(j, t):
            pltpu.async_copy(
                rows[t], out_hbm.at[pl.ds(base + j * ch, ch)], wsem[t]
            )

        def wait_w(j, t):
            pltpu.make_async_copy(
                rows[t], out_hbm.at[pl.ds(base + j * ch, ch)], wsem[t]
            ).wait()

        start_g(0, 0)
        start_g(1, 1)
        for j in range(2):
            wait_g(j, j % nbuf)
            start_w(j, j % nbuf)
            start_g(j + 2, (j + 2) % nbuf)
        for j in range(2, 4):
            wait_g(j, j % nbuf)
            start_w(j, j % nbuf)
            wait_w(j - 2, (j - 2) % nbuf)
            start_g(j + 2, (j + 2) % nbuf)

        def body(p, _):
            j0 = 4 * p
            for t in range(4):
                j = j0 + t
                b = (j0 + t) % nbuf
                wait_g(j, t)
                start_w(j, t)
                wait_w(j - 2, (t + 2) % nbuf)
                start_g(j + 2, (t + 2) % nbuf)
            return 0

        lax.fori_loop(1, n_ch // 4 - 1, body, 0)

        for j in range(n_ch - 4, n_ch):
            t = j % nbuf
            wait_g(j, t)
            start_w(j, t)
            wait_w(j - 2, (j - 2) % nbuf)
            if j + 2 < n_ch:
                start_g(j + 2, (j + 2) % nbuf)
        wait_w(n_ch - 2, (n_ch - 2) % nbuf)
        wait_w(n_ch - 1, (n_ch - 1) % nbuf)

    return gather


SPLITS = (2048, 1280, 768)
_row0s = tuple(sum(SPLITS[:i]) for i in range(len(SPLITS)))
_gathers = tuple(
    _make_gather(r0, n) for r0, n in zip(_row0s, SPLITS)
)


def kernel(inputs, centroids):
    flat_centroids = centroids.reshape(NCODEBOOKS * NCENTROIDS, SUBVECT)
    idx_parts = [
        _compute_codes_part(inputs, centroids, r0 // TB, n // TB)
        for r0, n in zip(_row0s, SPLITS)
    ]
    out_ref = jax.empty_ref(
        jax.ShapeDtypeStruct((BATCH * NCODEBOOKS, SUBVECT), jnp.float32)
    )
    for g, idx in zip(_gathers, idx_parts):
        g(flat_centroids, idx, out_ref)
    neighbour_idxs = jnp.concatenate(idx_parts, axis=0)
    rows = out_ref[...]
    outputs = rows.reshape(BATCH, NCODEBOOKS, SUBVECT)
    return (neighbour_idxs, outputs)


# speedup vs baseline: 1.0437x; 1.0437x over previous
import functools

import jax
import jax.numpy as jnp
from jax import lax
from jax.experimental import pallas as pl
from jax.experimental.pallas import tpu as pltpu
from jax.experimental.pallas import tpu_sc as plsc

NCENTROIDS = 1024
NCODEBOOKS = 16
SUBVECT = 256
BATCH = 4096

TB = 512
NB = BATCH // TB


def _mm_argmax_kernel(x_ref, cb_ref, out_ref):
    cols = []
    for c in range(NCODEBOOKS):
        resp = lax.dot_general(
            x_ref[:, c, :], cb_ref[c], (((1,), (1,)), ((), ())),
            preferred_element_type=jnp.float32,
        )
        code = jnp.argmax(resp, axis=1).astype(jnp.int32)
        cols.append((code + c * NCENTROIDS)[:, None])
    out_ref[...] = jnp.concatenate(cols, axis=1)


def _compute_codes_part(inputs, centroids, tile0, ntiles):
    return pl.pallas_call(
        _mm_argmax_kernel,
        grid=(ntiles,),
        in_specs=[
            pl.BlockSpec(
                (TB, NCODEBOOKS, SUBVECT), lambda b: (b + tile0, 0, 0)
            ),
            pl.BlockSpec(
                (NCODEBOOKS, NCENTROIDS, SUBVECT), lambda b: (0, 0, 0)
            ),
        ],
        out_specs=pl.BlockSpec((TB, NCODEBOOKS), lambda b: (b, 0)),
        out_shape=jax.ShapeDtypeStruct((ntiles * TB, NCODEBOOKS), jnp.int32),
    )(inputs, centroids)


def _make_gather(row0, nrows):
    info = plsc.get_sparse_core_info()
    nc, ns = info.num_cores, info.num_subcores
    nw = nc * ns
    rows_per_w = nrows // nw
    b_per_w = rows_per_w * NCODEBOOKS
    ch = 64
    n_ch = b_per_w // ch
    assert b_per_w % ch == 0 and n_ch % 4 == 0 and n_ch >= 8
    nbuf = 4
    mesh = plsc.VectorSubcoreMesh(core_axis_name="c", subcore_axis_name="s")

    @functools.partial(
        pl.kernel,
        mesh=mesh,
        out_type=(),
        scratch_types=[
            pltpu.VMEM((rows_per_w, NCODEBOOKS), jnp.int32),
            pltpu.VMEM((b_per_w,), jnp.int32),
            [pltpu.VMEM((ch, SUBVECT), jnp.float32) for _ in range(nbuf)],
            [pltpu.SemaphoreType.DMA for _ in range(nbuf)],
            [pltpu.SemaphoreType.DMA for _ in range(nbuf)],
        ],
    )
    def gather(table_hbm, idx_hbm, out_hbm, idx2d, idx_v, rows, gsem, wsem):
        wid = lax.axis_index("s") * nc + lax.axis_index("c")
        half_base = row0 * NCODEBOOKS
        pltpu.sync_copy(idx_hbm.at[pl.ds(wid * rows_per_w, rows_per_w), :], idx2d)

        def fl(r, _):
            idx_v[pl.ds(r * NCODEBOOKS, NCODEBOOKS)] = idx2d[r]
            return 0

        lax.fori_loop(0, rows_per_w, fl, 0)

        base = half_base + wid * b_per_w

        def start_g(j, t):
            pltpu.async_copy(
                table_hbm.at[idx_v.at[pl.ds(j * ch, ch)]], rows[t], gsem[t]
            )

        def wait_g(j, t):
            pltpu.make_async_copy(
                table_hbm.at[idx_v.at[pl.ds(j * ch, ch)]], rows[t], gsem[t]
            ).wait()

        def start_w(j, t):
            pltpu.async_copy(
                rows[t], out_hbm.at[pl.ds(base + j * ch, ch)], wsem[t]
            )

        def wait_w(j, t):
            pltpu.make_async_copy(
                rows[t], out_hbm.at[pl.ds(base + j * ch, ch)], wsem[t]
            ).wait()

        start_g(0, 0)
        start_g(1, 1)
        for j in range(2):
            wait_g(j, j % nbuf)
            start_w(j, j % nbuf)
            start_g(j + 2, (j + 2) % nbuf)
        for j in range(2, 4):
            wait_g(j, j % nbuf)
            start_w(j, j % nbuf)
            wait_w(j - 2, (j - 2) % nbuf)
            start_g(j + 2, (j + 2) % nbuf)

        def body(p, _):
            j0 = 4 * p
            for t in range(4):
                j = j0 + t
                b = (j0 + t) % nbuf
                wait_g(j, t)
                start_w(j, t)
                wait_w(j - 2, (t + 2) % nbuf)
                start_g(j + 2, (t + 2) % nbuf)
            return 0

        lax.fori_loop(1, n_ch // 4 - 1, body, 0)

        for j in range(n_ch - 4, n_ch):
            t = j % nbuf
            wait_g(j, t)
            start_w(j, t)
            wait_w(j - 2, (j - 2) % nbuf)
            if j + 2 < n_ch:
                start_g(j + 2, (j + 2) % nbuf)
        wait_w(n_ch - 2, (n_ch - 2) % nbuf)
        wait_w(n_ch - 1, (n_ch - 1) % nbuf)

    return gather


SPLIT = 2560
_gather0 = _make_gather(0, SPLIT)
_gather1 = _make_gather(SPLIT, BATCH - SPLIT)


def kernel(inputs, centroids):
    flat_centroids = centroids.reshape(NCODEBOOKS * NCENTROIDS, SUBVECT)
    idx_a = _compute_codes_part(inputs, centroids, 0, SPLIT // TB)
    idx_b = _compute_codes_part(inputs, centroids, SPLIT // TB,
                                (BATCH - SPLIT) // TB)
    out_ref = jax.empty_ref(
        jax.ShapeDtypeStruct((BATCH * NCODEBOOKS, SUBVECT), jnp.float32)
    )
    _gather0(flat_centroids, idx_a, out_ref)
    _gather1(flat_centroids, idx_b, out_ref)
    neighbour_idxs = jnp.concatenate([idx_a, idx_b], axis=0)
    rows = out_ref[...]
    outputs = rows.reshape(BATCH, NCODEBOOKS, SUBVECT)
    return (neighbour_idxs, outputs)
